# baseline (speedup 1.0000x reference)
import functools

import jax
import jax.numpy as jnp
from jax import lax
from jax.experimental import pallas as pl
from jax.experimental.pallas import tpu as pltpu
from jax.experimental.pallas import tpu_sc as plsc

DIM = 32
NC = 2
NS = 16
NW = NC * NS
LANES = 16
CHUNK = 128


def _emb_body(q_hbm, table_hbm, out_hbm, idx_v, rows_v, gsem, *, bpw):
    nchunk = bpw // CHUNK
    wid = lax.axis_index("s") * NC + lax.axis_index("c")
    base = wid * bpw
    pltpu.sync_copy(q_hbm.at[pl.ds(base, bpw)], idx_v)

    zeros = jnp.zeros((LANES,), jnp.float32)
    lane_iota = lax.iota(jnp.int32, LANES)

    def chunk_body(g, carry):
        cb = g * CHUNK
        pltpu.async_copy(table_hbm.at[idx_v.at[pl.ds(cb, CHUNK)]], rows_v, gsem).wait()

        def group_fix(j, c2):
            v = idx_v[pl.ds(cb + j * LANES, LANES)]
            npad = plsc.all_reduce_population_count(v == 0)
            has_pad = npad[0] > 0

            @pl.when(has_pad)
            def _():
                m = v == 0
                rows = lane_iota + j * LANES
                for k in range(DIM):
                    cols = jnp.full((LANES,), k, jnp.int32)
                    plsc.store_scatter(rows_v, [rows, cols], zeros, mask=m)

            return c2

        lax.fori_loop(0, CHUNK // LANES, group_fix, 0)
        pltpu.sync_copy(rows_v, out_hbm.at[pl.ds(base + cb, CHUNK)])
        return carry

    lax.fori_loop(0, nchunk, chunk_body, 0)


@functools.partial(jax.jit, static_argnames=("n",))
def _gather(table, qf, *, n):
    bpw = n // NW
    mesh = plsc.VectorSubcoreMesh(core_axis_name="c", subcore_axis_name="s")
    k = functools.partial(
        pl.kernel,
        mesh=mesh,
        out_type=jax.ShapeDtypeStruct((n, DIM), jnp.float32),
        compiler_params=pltpu.CompilerParams(
            needs_layout_passes=False, use_tc_tiling_on_sc=False
        ),
        scratch_types=[
            pltpu.VMEM((bpw,), jnp.int32),
            pltpu.VMEM((CHUNK, DIM), jnp.float32),
            pltpu.SemaphoreType.DMA,
        ],
    )(functools.partial(_emb_body, bpw=bpw))
    return k(qf, table)


def kernel(q, q_len, table):
    b, s = q.shape
    out = _gather(table, q.reshape(-1), n=b * s)
    return out.reshape(b, s, DIM)
